# Initial kernel scaffold; baseline (speedup 1.0000x reference)
import jax
import jax.numpy as jnp
from jax.experimental import pallas as pl


def kernel(x, edge_attr, edge_index, ptr, batch, nodes_sampled, log_probs, params):
    raise NotImplementedError("write your pallas kernel here")


# speedup vs baseline: 1.0152x; 1.0152x over previous
import jax
import jax.numpy as jnp
import numpy as np
from jax.experimental import pallas as pl

N = 10000; E = 160000; H = 128; M_SUB = 2; K_SUB = 8
G = 100; NPG = 100; RWSE_STEPS = 16; HEADS = 4
N_LAYERS = 6; R_LAYERS = 2; VOCAB = 128; EDIM = 16; FFN = 4 * H


def _layer_norm(v, g, b):
    mu = v.mean(-1, keepdims=True)
    var = ((v - mu) ** 2).mean(-1, keepdims=True)
    return (v - mu) / jnp.sqrt(var + 1e-5) * g + b


def _global_rwse(edge_index):
    src, dst = edge_index[0], edge_index[1]
    gs = src // NPG
    gd = dst // NPG
    same = (gs == gd)
    gidx = jnp.where(same, gs, 0)
    A = jnp.zeros((G, NPG, NPG), jnp.float32).at[gidx, src % NPG, dst % NPG].add(same.astype(jnp.float32))
    deg = A.sum(-1, keepdims=True)
    P = A / jnp.maximum(deg, 1.0)
    Mt = P
    diags = []
    for _ in range(RWSE_STEPS):
        diags.append(jnp.diagonal(Mt, axis1=1, axis2=2))
        Mt = Mt @ P
    return jnp.stack(diags, axis=-1).reshape(G * NPG, RWSE_STEPS)


def _seg_sum_kernel(ne_ref, out_ref):
    out_ref[...] = ne_ref[...].reshape(G, NPG, H).sum(axis=1)


def kernel(x, edge_attr, edge_index, ptr, batch, nodes_sampled, log_probs, params):
    p = params
    x_emb = p['atom_emb'][x[:, 0]]
    ea_emb = p['bond_emb'][edge_attr[:, 0] - 1]
    S, k = nodes_sampled.shape
    node_ids = nodes_sampled.reshape(-1)
    nid = node_ids
    x_flat = x_emb[nid]
    base = jnp.arange(S, dtype=jnp.int32) * k
    leaves = (base[:, None] + jnp.arange(1, k, dtype=jnp.int32)[None, :]).reshape(-1)
    roots_rep = jnp.repeat(base, k - 1)
    e_src = jnp.concatenate([roots_rep, leaves])
    e_dst = jnp.concatenate([leaves, roots_rep])
    ea_flat = ea_emb[jnp.arange(e_src.shape[0]) % E]
    root_flat_idx = base
    lp = log_probs
    rwse = _global_rwse(edge_index)
    rwse_flat = jax.nn.relu(rwse @ p['rwse_W'] + p['rwse_b'])[nid]
    h = x_flat + rwse_flat
    for lyr in p['layers']:
        msg = jax.nn.relu(h[e_src] + ea_flat)
        agg = jnp.zeros_like(h).at[e_dst].add(msg)
        hi = (1.0 + lyr['eps']) * h + agg
        hi = jax.nn.relu(hi @ lyr['W1'] + lyr['b1']) @ lyr['W2'] + lyr['b2']
        cnt = jnp.zeros((N,), jnp.float32).at[nid].add(1.0)
        gpool = jnp.zeros((N, H), jnp.float32).at[nid].add(hi) / jnp.maximum(cnt, 1.0)[:, None]
        gm = jnp.zeros((N, H), jnp.float32).at[edge_index[1]].add(jax.nn.relu(gpool[edge_index[0]] + ea_emb))
        h = h + (hi + (gm @ lyr['Wg'] + lyr['bg'])[nid])
    h_tok = h[root_flat_idx].reshape(N, M_SUB, H)
    lp2 = lp.reshape(N, M_SUB)
    bias = p['ht_alpha'][0] * jnp.broadcast_to(lp2[:, None, :], (N, M_SUB, M_SUB))
    bias_h = jnp.broadcast_to(bias[:, None, :, :], (N, HEADS, M_SUB, M_SUB))
    dh = H // HEADS
    for rl in p['readout']:
        xn = _layer_norm(h_tok, rl['ln1_g'], rl['ln1_b'])
        qkv = xn @ rl['Wqkv'] + rl['bqkv']
        q, kk, v = jnp.split(qkv, 3, axis=-1)
        q = q.reshape(N, M_SUB, HEADS, dh).transpose(0, 2, 1, 3)
        kk = kk.reshape(N, M_SUB, HEADS, dh).transpose(0, 2, 1, 3)
        v = v.reshape(N, M_SUB, HEADS, dh).transpose(0, 2, 1, 3)
        att = (q @ kk.transpose(0, 1, 3, 2)) / np.sqrt(dh) + bias_h
        att = jax.nn.softmax(att, axis=-1)
        o = (att @ v).transpose(0, 2, 1, 3).reshape(N, M_SUB, H)
        h_tok = h_tok + (o @ rl['Wo'] + rl['bo'])
        xn2 = _layer_norm(h_tok, rl['ln2_g'], rl['ln2_b'])
        h_tok = h_tok + (jax.nn.gelu(xn2 @ rl['Wf1'] + rl['bf1']) @ rl['Wf2'] + rl['bf2'])
    node_emb = _layer_norm(h_tok.mean(axis=1), p['ro_g'], p['ro_b'])
    return pl.pallas_call(
        _seg_sum_kernel,
        out_shape=jax.ShapeDtypeStruct((G, H), jnp.float32),
    )(node_emb)


# speedup vs baseline: 3.2878x; 3.2386x over previous
import functools

import jax
import jax.numpy as jnp
import numpy as np
from jax import lax
from jax.experimental import pallas as pl
from jax.experimental.pallas import tpu as pltpu, tpu_sc as plsc

N = 10000; E = 160000; H = 128; M_SUB = 2; K_SUB = 8
G = 100; NPG = 100; RWSE_STEPS = 16; HEADS = 4
N_LAYERS = 6; R_LAYERS = 2; VOCAB = 128; EDIM = 16; FFN = 4 * H
S = N * M_SUB
R = S * K_SUB
NW = 32
CH = 128
NCHUNK = R // CH
NIT = (NCHUNK + NW - 1) // NW
APAD = 112
AN = G * NPG
NPAD = 10240
NP_SUB = NPAD // 16
ASZ = 1 << 20
B_SG = 250
NBLK = S // B_SG
DH = H // HEADS

@functools.cache
def _mesh():
    return plsc.VectorSubcoreMesh(core_axis_name="c", subcore_axis_name="s",
                                  num_cores=2, num_subcores=16)


def _wid():
    return lax.axis_index("s") * 2 + lax.axis_index("c")


def _sc_gather_body(table, idx, out, idxv, rows, sem):
    w = _wid()

    def step(i, carry):
        ch = i * NW + w

        @pl.when(ch < NCHUNK)
        def _():
            base = ch * CH
            pltpu.sync_copy(idx.at[pl.ds(base, CH)], idxv.at[0])
            pltpu.async_copy(table.at[idxv.at[0]], rows, sem).wait()
            pltpu.sync_copy(rows, out.at[pl.ds(base, CH)])

        return carry

    lax.fori_loop(0, NIT, step, 0)


def _sc_gather(table, idx):
    return pl.kernel(
        _sc_gather_body,
        out_type=jax.ShapeDtypeStruct((R, H), jnp.float32),
        mesh=_mesh(),
        scratch_types=[
            pltpu.VMEM((1, CH), jnp.int32),
            pltpu.VMEM((CH, H), jnp.float32),
            pltpu.SemaphoreType.DMA,
        ],
    )(table, idx)


def _sc_scatter_body(vals, idx, zeros, out, idxv, rows, acc):
    c = lax.axis_index("c")
    s = lax.axis_index("s")
    w = s * 2 + c
    r0 = s * NP_SUB
    pltpu.sync_copy(zeros.at[pl.ds(r0, NP_SUB)], acc.at[pl.ds(r0, NP_SUB)])
    plsc.subcore_barrier()

    def step(i, carry):
        ch = i * NW + w

        @pl.when(ch < NCHUNK)
        def _():
            base = ch * CH
            pltpu.sync_copy(idx.at[pl.ds(base, CH)], idxv.at[0])
            pltpu.sync_copy(vals.at[pl.ds(base, CH)], rows)
            pltpu.sync_copy(rows, acc.at[idxv.at[0]], add=True)

        return carry

    lax.fori_loop(0, NIT, step, 0)
    plsc.subcore_barrier()
    pltpu.sync_copy(acc.at[pl.ds(r0, NP_SUB)],
                    out.at[pl.ds(c * NPAD + r0, NP_SUB)])


def _sc_scatter(vals, idx, zeros):
    return pl.kernel(
        _sc_scatter_body,
        out_type=jax.ShapeDtypeStruct((2 * NPAD, H), jnp.float32),
        mesh=_mesh(),
        scratch_types=[
            pltpu.VMEM((1, CH), jnp.int32),
            pltpu.VMEM((CH, H), jnp.float32),
            pltpu.VMEM_SHARED((NPAD, H), jnp.float32),
        ],
    )(vals, idx, zeros)


def _sc_cnt_body(idx, zeros, ones, out, idxv, rows, acc):
    c = lax.axis_index("c")
    s = lax.axis_index("s")
    w = s * 2 + c
    r0 = s * NP_SUB
    pltpu.sync_copy(zeros.at[pl.ds(r0, NP_SUB)], acc.at[pl.ds(r0, NP_SUB)])
    pltpu.sync_copy(ones, rows)
    plsc.subcore_barrier()

    def step(i, carry):
        ch = i * NW + w

        @pl.when(ch < NCHUNK)
        def _():
            base = ch * CH
            pltpu.sync_copy(idx.at[pl.ds(base, CH)], idxv.at[0])
            pltpu.sync_copy(rows, acc.at[idxv.at[0]], add=True)

        return carry

    lax.fori_loop(0, NIT, step, 0)
    plsc.subcore_barrier()
    pltpu.sync_copy(acc.at[pl.ds(r0, NP_SUB)],
                    out.at[pl.ds(c * NPAD + r0, NP_SUB)])


def _sc_cnt(idx, zeros, ones):
    return pl.kernel(
        _sc_cnt_body,
        out_type=jax.ShapeDtypeStruct((2 * NPAD, H), jnp.float32),
        mesh=_mesh(),
        scratch_types=[
            pltpu.VMEM((1, CH), jnp.int32),
            pltpu.VMEM((CH, H), jnp.float32),
            pltpu.VMEM_SHARED((NPAD, H), jnp.float32),
        ],
    )(idx, zeros, ones)


def _sc_build_a_body(src, dst, za, out, sv, dv, posv, valv, acc):
    c = lax.axis_index("c")
    s = lax.axis_index("s")
    w = s * 2 + c
    r0 = s * (ASZ // 16)
    pltpu.sync_copy(za.at[pl.ds(r0, ASZ // 16)], acc.at[pl.ds(r0, ASZ // 16)])
    plsc.subcore_barrier()

    def step(i, carry):
        ch = i * NW + w

        @pl.when(ch < NCHUNK)
        def _():
            base = ch * CH
            pltpu.sync_copy(src.at[pl.ds(base, CH)], sv.at[0])
            pltpu.sync_copy(dst.at[pl.ds(base, CH)], dv.at[0])
            for j in range(CH // 16):
                s16 = sv[0, pl.ds(j * 16, 16)]
                d16 = dv[0, pl.ds(j * 16, 16)]
                srcm = s16 % NPG
                dstm = d16 % NPG
                same = (s16 - srcm) == (d16 - dstm)
                row = jnp.where(same, s16 - srcm,
                                jnp.zeros((16,), jnp.int32)) + srcm
                val = jnp.where(same, jnp.ones((16,), jnp.float32),
                                jnp.zeros((16,), jnp.float32))
                posv[0, pl.ds(j * 16, 16)] = row * NPG + dstm
                valv[0, pl.ds(j * 16, 16)] = val
            pltpu.sync_copy(valv.at[0], acc.at[posv.at[0]], add=True)

        return carry

    lax.fori_loop(0, NIT, step, 0)
    plsc.subcore_barrier()
    pltpu.sync_copy(acc.at[pl.ds(r0, ASZ // 16)],
                    out.at[pl.ds(c * ASZ + r0, ASZ // 16)])


def _sc_build_a(src, dst, za):
    return pl.kernel(
        _sc_build_a_body,
        out_type=jax.ShapeDtypeStruct((2 * ASZ,), jnp.float32),
        mesh=_mesh(),
        scratch_types=[
            pltpu.VMEM((1, CH), jnp.int32),
            pltpu.VMEM((1, CH), jnp.int32),
            pltpu.VMEM((1, CH), jnp.int32),
            pltpu.VMEM((1, CH), jnp.float32),
            pltpu.VMEM_SHARED((ASZ,), jnp.float32),
        ],
    )(src, dst, za)


def _onehot_rows(ids, width):
    cols = lax.broadcasted_iota(jnp.int32, (1, width), 1)
    return (ids[:, None] == cols).astype(jnp.float32)


def _rwse_body(a0, a1, xids, atom, rw_w, rw_b, out):
    a = a0[0] + a1[0]
    deg = jnp.sum(a, axis=1, keepdims=True)
    p = a / jnp.maximum(deg, 1.0)
    ri = lax.broadcasted_iota(jnp.int32, (NPG, NPG), 0)
    ci = lax.broadcasted_iota(jnp.int32, (NPG, NPG), 1)
    eye = (ri == ci).astype(jnp.float32)
    mt = p
    diags = []
    for _ in range(RWSE_STEPS):
        diags.append(jnp.sum(mt * eye, axis=1, keepdims=True))
        mt = jnp.dot(mt, p, preferred_element_type=jnp.float32)
    rw = jnp.concatenate(diags, axis=1)
    rwf = jnp.maximum(jnp.dot(rw, rw_w[...], preferred_element_type=jnp.float32)
                      + rw_b[...], 0.0)
    oh = _onehot_rows(xids[0, 0, :], VOCAB)
    out[0] = rwf + jnp.dot(oh, atom[...], preferred_element_type=jnp.float32)


def _tc_rwse(a0, a1, xids, atom, rw_w, rw_b):
    return pl.pallas_call(
        _rwse_body,
        grid=(G,),
        in_specs=[
            pl.BlockSpec((1, NPG, NPG), lambda i: (i, 0, 0)),
            pl.BlockSpec((1, NPG, NPG), lambda i: (i, 0, 0)),
            pl.BlockSpec((1, 1, NPG), lambda i: (i, 0, 0)),
            pl.BlockSpec((VOCAB, H), lambda i: (0, 0)),
            pl.BlockSpec((RWSE_STEPS, H), lambda i: (0, 0)),
            pl.BlockSpec((1, H), lambda i: (0, 0)),
        ],
        out_specs=pl.BlockSpec((1, NPG, H), lambda i: (i, 0, 0)),
        out_shape=jax.ShapeDtypeStruct((G, NPG, H), jnp.float32),
    )(a0, a1, xids, atom, rw_w, rw_b).reshape(N, H)


def _cinv_body(c0, c1, out):
    out[...] = 1.0 / jnp.maximum(c0[...] + c1[...], 1.0)


def _tc_cinv(c0, c1):
    return pl.pallas_call(
        _cinv_body,
        grid=(5,),
        in_specs=[pl.BlockSpec((N // 5, H), lambda i: (i, 0))] * 2,
        out_specs=pl.BlockSpec((N // 5, H), lambda i: (i, 0)),
        out_shape=jax.ShapeDtypeStruct((N, H), jnp.float32),
    )(c0, c1)


def _mlp(v, w1, b1, w2, b2):
    a = jnp.maximum(jnp.dot(v, w1, preferred_element_type=jnp.float32) + b1, 0.0)
    return jnp.dot(a, w2, preferred_element_type=jnp.float32) + b2


def _make_layer_body(first, last):
    def body(*refs):
        if first:
            (h_ref, ea1_ref, ea2_ref, bond_ref,
             w1r, b1r, w2r, b2r, epsr, *outs) = refs
            h3 = h_ref[...]
        else:
            (h_ref, hi_ref, gwf_ref, ea1_ref, ea2_ref, bond_ref,
             w1r, b1r, w2r, b2r, epsr, *outs) = refs
            h3 = h_ref[...] + hi_ref[...] + gwf_ref[...]
        if last:
            hi_out, h6r_out = outs
        elif first:
            (hi_out,) = outs
        else:
            hi_out, h_out = outs
            h_out[...] = h3
        bond = bond_ref[...]
        ea1 = jnp.dot(_onehot_rows(ea1_ref[0, 0, :] - 1, EDIM), bond,
                      preferred_element_type=jnp.float32).reshape(B_SG, K_SUB - 1, H)
        ea2 = jnp.dot(_onehot_rows(ea2_ref[0, 0, :] - 1, EDIM), bond,
                      preferred_element_type=jnp.float32).reshape(B_SG, K_SUB - 1, H)
        root = h3[:, 0:1, :]
        leaves = h3[:, 1:, :]
        eps1 = 1.0 + epsr[0, 0]
        star_root = eps1 * h3[:, 0, :] + jnp.sum(
            jnp.maximum(leaves + ea2, 0.0), axis=1)
        star_leaf = eps1 * leaves + jnp.maximum(root + ea1, 0.0)
        w1, b1, w2, b2 = w1r[...], b1r[...], w2r[...], b2r[...]
        hi_root = _mlp(star_root, w1, b1, w2, b2)
        hi_leaf = _mlp(star_leaf.reshape(B_SG * (K_SUB - 1), H),
                       w1, b1, w2, b2).reshape(B_SG, K_SUB - 1, H)
        hi_out[:, 0:1, :] = hi_root[:, None, :]
        hi_out[:, 1:, :] = hi_leaf
        if last:
            h6r_out[0] = h3[:, 0, :] + hi_root

    return body


def _tc_layer(h3, hi3, gwf3, ea1_ids, ea2_ids, bond, w1, b1, w2, b2, eps1,
              first, last):
    blk3 = pl.BlockSpec((B_SG, K_SUB, H), lambda i: (i, 0, 0))
    ids_spec = pl.BlockSpec((1, 1, B_SG * (K_SUB - 1)), lambda i: (i, 0, 0))
    w_specs = [
        pl.BlockSpec((EDIM, H), lambda i: (0, 0)),
        pl.BlockSpec((H, H), lambda i: (0, 0)),
        pl.BlockSpec((1, H), lambda i: (0, 0)),
        pl.BlockSpec((H, H), lambda i: (0, 0)),
        pl.BlockSpec((1, H), lambda i: (0, 0)),
        pl.BlockSpec(memory_space=pltpu.SMEM),
    ]
    hi_shape = jax.ShapeDtypeStruct((S, K_SUB, H), jnp.float32)
    if first:
        in_specs = [blk3, ids_spec, ids_spec] + w_specs
        args = (h3, ea1_ids, ea2_ids, bond, w1, b1, w2, b2, eps1)
        out_specs, out_shape = (blk3,), (hi_shape,)
    else:
        in_specs = [blk3, blk3, blk3, ids_spec, ids_spec] + w_specs
        args = (h3, hi3, gwf3, ea1_ids, ea2_ids, bond, w1, b1, w2, b2, eps1)
        if last:
            out_specs = (blk3, pl.BlockSpec((1, B_SG, H), lambda i: (i, 0, 0)))
            out_shape = (hi_shape,
                         jax.ShapeDtypeStruct((NBLK, B_SG, H), jnp.float32))
        else:
            out_specs, out_shape = (blk3, blk3), (hi_shape, hi_shape)
    return pl.pallas_call(
        _make_layer_body(first, last),
        grid=(NBLK,),
        in_specs=in_specs,
        out_specs=list(out_specs),
        out_shape=list(out_shape),
    )(*args)


def _gpool_body(g0, g1, cinv, out):
    out[...] = (g0[...] + g1[...]) * cinv[...]


def _tc_gpool(g0, g1, cinv):
    return pl.pallas_call(
        _gpool_body,
        grid=(5,),
        in_specs=[pl.BlockSpec((N // 5, H), lambda i: (i, 0))] * 3,
        out_specs=pl.BlockSpec((N // 5, H), lambda i: (i, 0)),
        out_shape=jax.ShapeDtypeStruct((N, H), jnp.float32),
    )(g0, g1, cinv)


def _me_body(ge, ids_ref, bond, out):
    ea = jnp.dot(_onehot_rows(ids_ref[0, 0, :] - 1, EDIM), bond[...],
                 preferred_element_type=jnp.float32)
    out[...] = jnp.maximum(ge[...] + ea, 0.0)


def _tc_me(ge, me_ids, bond):
    blk = E // NBLK
    return pl.pallas_call(
        _me_body,
        grid=(NBLK,),
        in_specs=[
            pl.BlockSpec((blk, H), lambda i: (i, 0)),
            pl.BlockSpec((1, 1, blk), lambda i: (i, 0, 0)),
            pl.BlockSpec((EDIM, H), lambda i: (0, 0)),
        ],
        out_specs=pl.BlockSpec((blk, H), lambda i: (i, 0)),
        out_shape=jax.ShapeDtypeStruct((E, H), jnp.float32),
    )(ge, me_ids, bond)


def _gw_body(g0, g1, wg, bg, out):
    out[...] = jnp.dot(g0[...] + g1[...], wg[...],
                       preferred_element_type=jnp.float32) + bg[...]


def _tc_gw(g0, g1, wg, bg):
    return pl.pallas_call(
        _gw_body,
        grid=(5,),
        in_specs=[
            pl.BlockSpec((N // 5, H), lambda i: (i, 0)),
            pl.BlockSpec((N // 5, H), lambda i: (i, 0)),
            pl.BlockSpec((H, H), lambda i: (0, 0)),
            pl.BlockSpec((1, H), lambda i: (0, 0)),
        ],
        out_specs=pl.BlockSpec((N // 5, H), lambda i: (i, 0)),
        out_shape=jax.ShapeDtypeStruct((N, H), jnp.float32),
    )(g0, g1, wg, bg)


def _layer_norm(v, g, b):
    mu = jnp.mean(v, axis=1, keepdims=True)
    var = jnp.mean((v - mu) ** 2, axis=1, keepdims=True)
    return (v - mu) / jnp.sqrt(var + 1e-5) * g + b


def _readout_body(h6r, gw6, lp, alpha, *rest):
    (l1g_a, l1b_a, wqkv_a, bqkv_a, wo_a, bo_a, l2g_a, l2b_a,
     wf1_a, bf1_a, wf2_a, bf2_a,
     l1g_b, l1b_b, wqkv_b, bqkv_b, wo_b, bo_b, l2g_b, l2b_b,
     wf1_b, bf1_b, wf2_b, bf2_b, ro_g, ro_b, out) = rest
    bn = h6r.shape[0]
    g = gw6[...]
    t0 = h6r[:, 0, :] + g
    t1 = h6r[:, 1, :] + g
    al = alpha[0, 0]
    lpv = lp[0]
    lp0 = al * lpv[:, 0:1]
    lp1 = al * lpv[:, 1:2]
    hri = lax.broadcasted_iota(jnp.int32, (H, HEADS), 0)
    hci = lax.broadcasted_iota(jnp.int32, (H, HEADS), 1)
    hs = (hri // DH == hci).astype(jnp.float32)
    he = hs.T
    inv = 1.0 / np.sqrt(DH)
    for (l1g, l1b, wqkv, bqkv, wo, bo, l2g, l2b, wf1, bf1, wf2, bf2) in (
        (l1g_a, l1b_a, wqkv_a, bqkv_a, wo_a, bo_a, l2g_a, l2b_a,
         wf1_a, bf1_a, wf2_a, bf2_a),
        (l1g_b, l1b_b, wqkv_b, bqkv_b, wo_b, bo_b, l2g_b, l2b_b,
         wf1_b, bf1_b, wf2_b, bf2_b),
    ):
        xn0 = _layer_norm(t0, l1g[...], l1b[...])
        xn1 = _layer_norm(t1, l1g[...], l1b[...])
        qkv0 = jnp.dot(xn0, wqkv[...], preferred_element_type=jnp.float32) + bqkv[...]
        qkv1 = jnp.dot(xn1, wqkv[...], preferred_element_type=jnp.float32) + bqkv[...]
        q0, k0, v0 = qkv0[:, :H], qkv0[:, H:2 * H], qkv0[:, 2 * H:]
        q1, k1, v1 = qkv1[:, :H], qkv1[:, H:2 * H], qkv1[:, 2 * H:]

        def scores(q, k, lpj):
            return jnp.dot(q * k, hs, preferred_element_type=jnp.float32) * inv + lpj

        s00 = scores(q0, k0, lp0)
        s01 = scores(q0, k1, lp1)
        s10 = scores(q1, k0, lp0)
        s11 = scores(q1, k1, lp1)

        def soft(sa, sb):
            m = jnp.maximum(sa, sb)
            ea = jnp.exp(sa - m)
            eb = jnp.exp(sb - m)
            d = ea + eb
            return ea / d, eb / d

        w00, w01 = soft(s00, s01)
        w10, w11 = soft(s10, s11)

        def expand(w):
            return jnp.dot(w, he, preferred_element_type=jnp.float32)

        o0 = expand(w00) * v0 + expand(w01) * v1
        o1 = expand(w10) * v0 + expand(w11) * v1
        t0 = t0 + jnp.dot(o0, wo[...], preferred_element_type=jnp.float32) + bo[...]
        t1 = t1 + jnp.dot(o1, wo[...], preferred_element_type=jnp.float32) + bo[...]
        xn20 = _layer_norm(t0, l2g[...], l2b[...])
        xn21 = _layer_norm(t1, l2g[...], l2b[...])
        t0 = t0 + jnp.dot(jax.nn.gelu(
            jnp.dot(xn20, wf1[...], preferred_element_type=jnp.float32) + bf1[...]),
            wf2[...], preferred_element_type=jnp.float32) + bf2[...]
        t1 = t1 + jnp.dot(jax.nn.gelu(
            jnp.dot(xn21, wf1[...], preferred_element_type=jnp.float32) + bf1[...]),
            wf2[...], preferred_element_type=jnp.float32) + bf2[...]
    ne = _layer_norm((t0 + t1) * 0.5, ro_g[...], ro_b[...])
    out[0] = jnp.sum(ne.reshape(bn // NPG, NPG, H), axis=1)


def _tc_readout(h6r3, gw6, lp3, alpha, ro_params, ro_g, ro_b):
    bn = 1000
    ngrid = N // bn
    full = lambda shp: pl.BlockSpec(shp, lambda i: tuple(0 for _ in shp))
    in_specs = [
        pl.BlockSpec((bn, M_SUB, H), lambda i: (i, 0, 0)),
        pl.BlockSpec((bn, H), lambda i: (i, 0)),
        pl.BlockSpec((1, bn, M_SUB), lambda i: (i, 0, 0)),
        pl.BlockSpec(memory_space=pltpu.SMEM),
    ]
    args = [h6r3, gw6, lp3, alpha]
    for rl in ro_params:
        for nm, shp in (("ln1_g", (1, H)), ("ln1_b", (1, H)),
                        ("Wqkv", (H, 3 * H)), ("bqkv", (1, 3 * H)),
                        ("Wo", (H, H)), ("bo", (1, H)),
                        ("ln2_g", (1, H)), ("ln2_b", (1, H)),
                        ("Wf1", (H, FFN)), ("bf1", (1, FFN)),
                        ("Wf2", (FFN, H)), ("bf2", (1, H))):
            in_specs.append(full(shp))
            args.append(rl[nm].reshape(shp))
    in_specs += [full((1, H)), full((1, H))]
    args += [ro_g.reshape(1, H), ro_b.reshape(1, H)]
    return pl.pallas_call(
        _readout_body,
        grid=(ngrid,),
        in_specs=in_specs,
        out_specs=pl.BlockSpec((1, bn // NPG, H), lambda i: (i, 0, 0)),
        out_shape=jax.ShapeDtypeStruct((ngrid, bn // NPG, H), jnp.float32),
    )(*args).reshape(G, H)


def kernel(x, edge_attr, edge_index, ptr, batch, nodes_sampled, log_probs, params):
    p = params
    nid = nodes_sampled.reshape(-1)
    ea_ids = edge_attr[:, 0]
    n_star = S * (K_SUB - 1)
    ea1_ids = ea_ids[:n_star].reshape(NBLK, 1, n_star // NBLK)
    ea2_flat = jnp.concatenate([ea_ids[n_star:], ea_ids[:n_star - (E - n_star)]])
    ea2_ids = ea2_flat.reshape(NBLK, 1, n_star // NBLK)
    me_ids = ea_ids.reshape(NBLK, 1, E // NBLK)
    src = edge_index[0]
    dst = edge_index[1]
    xids = x[:, 0].reshape(G, 1, NPG)
    lp3 = log_probs.reshape(N // 100, 100, M_SUB).reshape(10, 1000, M_SUB)
    zeros_nh = jnp.zeros((NPAD, H), jnp.float32)
    zeros_a = jnp.zeros((ASZ,), jnp.float32)
    ones_ch = jnp.ones((CH, H), jnp.float32)
    bond = p['bond_emb']

    ap = _sc_build_a(src, dst, zeros_a)
    cntp = _sc_cnt(nid, zeros_nh, ones_ch)
    cinv = _tc_cinv(cntp[:N], cntp[NPAD:NPAD + N])
    table0 = _tc_rwse(ap[:AN * NPG].reshape(G, NPG, NPG),
                      ap[ASZ:ASZ + AN * NPG].reshape(G, NPG, NPG),
                      xids, p['atom_emb'],
                      p['rwse_W'], p['rwse_b'].reshape(1, H))
    h3 = _sc_gather(table0, nid).reshape(S, K_SUB, H)

    hi3 = None
    gwf3 = None
    h6r = None
    gw = None
    for l in range(N_LAYERS):
        lyr = p['layers'][l]
        eps1 = lyr['eps'].reshape(1, 1)
        first = (l == 0)
        last = (l == N_LAYERS - 1)
        outs = _tc_layer(h3, hi3, gwf3, ea1_ids, ea2_ids, bond,
                         lyr['W1'], lyr['b1'].reshape(1, H),
                         lyr['W2'], lyr['b2'].reshape(1, H), eps1, first, last)
        if first:
            (hi_l,) = outs
        elif last:
            hi_l, h6r = outs
        else:
            hi_l, h3 = outs
        gsp = _sc_scatter(hi_l.reshape(R, H), nid, zeros_nh)
        gpool = _tc_gpool(gsp[:N], gsp[NPAD:NPAD + N], cinv)
        gep = _sc_gather(gpool, src)
        me = _tc_me(gep, me_ids, bond)
        gmp = _sc_scatter(me, dst, zeros_nh)
        gw = _tc_gw(gmp[:N], gmp[NPAD:NPAD + N], lyr['Wg'],
                    lyr['bg'].reshape(1, H))
        if not last:
            gwf3 = _sc_gather(gw, nid).reshape(S, K_SUB, H)
            hi3 = hi_l

    return _tc_readout(h6r.reshape(N, M_SUB, H), gw, lp3,

                       p['ht_alpha'].reshape(1, 1), p['readout'],
                       p['ro_g'], p['ro_b'])


# speedup vs baseline: 4.0572x; 1.2340x over previous
import functools

import jax
import jax.numpy as jnp
import numpy as np
from jax import lax
from jax.experimental import pallas as pl
from jax.experimental.pallas import tpu as pltpu, tpu_sc as plsc

N = 10000; E = 160000; H = 128; M_SUB = 2; K_SUB = 8
G = 100; NPG = 100; RWSE_STEPS = 16; HEADS = 4
N_LAYERS = 6; R_LAYERS = 2; VOCAB = 128; EDIM = 16; FFN = 4 * H
S = N * M_SUB
R = S * K_SUB
NW = 32
CH = 128
NCHUNK = R // CH
NIT = (NCHUNK + NW - 1) // NW
APAD = 112
AN = G * NPG
NPAD = 10240
NP_SUB = NPAD // 16
ASZ = 1 << 20
B_SG = 250
NBLK = S // B_SG
DH = H // HEADS

@functools.cache
def _mesh():
    return plsc.VectorSubcoreMesh(core_axis_name="c", subcore_axis_name="s",
                                  num_cores=2, num_subcores=16)


def _wid():
    return lax.axis_index("s") * 2 + lax.axis_index("c")


CPW = NCHUNK // NW
REM = NCHUNK - CPW * NW
NPAIR = (CPW + 2) // 2


def _load_idx_strip(idx2, idxv, w):
    base_ch = w * CPW
    pltpu.sync_copy(idx2.at[pl.ds(base_ch, CPW)], idxv.at[pl.ds(0, CPW)])

    @pl.when(w < REM)
    def _():
        pltpu.sync_copy(idx2.at[pl.ds(NW * CPW + w, 1)],
                        idxv.at[pl.ds(CPW, 1)])


def _idx_row(idxv, j):
    return idxv.at[j, 0]


def _gchunk(w, j):
    return jnp.where(j < CPW, w * CPW + j, NW * CPW + w)


def _sc_gather_body(table, idx2, out, idxv, b0, b1, s0, s1):
    w = _wid()
    _load_idx_strip(idx2, idxv, w)
    nch = CPW + jnp.where(w < REM, 1, 0)

    def issue(j, buf, sem):
        pltpu.async_copy(table.at[_idx_row(idxv, j)], buf, sem)

    def drain(j, buf, sem):
        pltpu.make_async_copy(table.at[pl.ds(0, CH)], buf, sem).wait()
        pltpu.sync_copy(buf, out.at[pl.ds(_gchunk(w, j) * CH, CH)])

    issue(jnp.int32(0), b0, s0)

    def pair(k, carry):
        j0 = 2 * k
        j1 = j0 + 1

        @pl.when(j1 < nch)
        def _():
            issue(j1, b1, s1)

        @pl.when(j0 < nch)
        def _():
            drain(j0, b0, s0)

        @pl.when(j0 + 2 < nch)
        def _():
            issue(j0 + 2, b0, s0)

        @pl.when(j1 < nch)
        def _():
            drain(j1, b1, s1)

        return carry

    lax.fori_loop(0, NPAIR, pair, 0)


def _sc_gather(table, idx2):
    return pl.kernel(
        _sc_gather_body,
        out_type=jax.ShapeDtypeStruct((R, H), jnp.float32),
        mesh=_mesh(),
        scratch_types=[
            pltpu.VMEM((CPW + 1, 1, CH), jnp.int32),
            pltpu.VMEM((CH, H), jnp.float32),
            pltpu.VMEM((CH, H), jnp.float32),
            pltpu.SemaphoreType.DMA,
            pltpu.SemaphoreType.DMA,
        ],
    )(table, idx2)


def _sc_scatter_body(vals, idx2, zeros, out, idxv, b0, b1, s0, s1, acc):
    c = lax.axis_index("c")
    s = lax.axis_index("s")
    w = s * 2 + c
    r0 = s * NP_SUB
    pltpu.sync_copy(zeros.at[pl.ds(r0, NP_SUB)], acc.at[pl.ds(r0, NP_SUB)])
    _load_idx_strip(idx2, idxv, w)
    plsc.subcore_barrier()
    nch = CPW + jnp.where(w < REM, 1, 0)

    def issue(j, buf, sem):
        pltpu.async_copy(vals.at[pl.ds(_gchunk(w, j) * CH, CH)], buf, sem)

    def drain(j, buf, sem):
        pltpu.make_async_copy(vals.at[pl.ds(0, CH)], buf, sem).wait()
        pltpu.sync_copy(buf, acc.at[_idx_row(idxv, j)], add=True)

    issue(jnp.int32(0), b0, s0)

    def pair(k, carry):
        j0 = 2 * k
        j1 = j0 + 1

        @pl.when(j1 < nch)
        def _():
            issue(j1, b1, s1)

        @pl.when(j0 < nch)
        def _():
            drain(j0, b0, s0)

        @pl.when(j0 + 2 < nch)
        def _():
            issue(j0 + 2, b0, s0)

        @pl.when(j1 < nch)
        def _():
            drain(j1, b1, s1)

        return carry

    lax.fori_loop(0, NPAIR, pair, 0)
    plsc.subcore_barrier()
    pltpu.sync_copy(acc.at[pl.ds(r0, NP_SUB)],
                    out.at[pl.ds(c * NPAD + r0, NP_SUB)])


def _sc_scatter(vals, idx2, zeros):
    return pl.kernel(
        _sc_scatter_body,
        out_type=jax.ShapeDtypeStruct((2 * NPAD, H), jnp.float32),
        mesh=_mesh(),
        scratch_types=[
            pltpu.VMEM((CPW + 1, 1, CH), jnp.int32),
            pltpu.VMEM((CH, H), jnp.float32),
            pltpu.VMEM((CH, H), jnp.float32),
            pltpu.SemaphoreType.DMA,
            pltpu.SemaphoreType.DMA,
            pltpu.VMEM_SHARED((NPAD, H), jnp.float32),
        ],
    )(vals, idx2, zeros)


def _sc_cnt_body(idx, zeros, ones, out, idxv, rows, acc):
    c = lax.axis_index("c")
    s = lax.axis_index("s")
    w = s * 2 + c
    r0 = s * NP_SUB
    pltpu.sync_copy(zeros.at[pl.ds(r0, NP_SUB)], acc.at[pl.ds(r0, NP_SUB)])
    pltpu.sync_copy(ones, rows)
    plsc.subcore_barrier()

    def step(i, carry):
        ch = i * NW + w

        @pl.when(ch < NCHUNK)
        def _():
            base = ch * CH
            pltpu.sync_copy(idx.at[pl.ds(base, CH)], idxv.at[0])
            pltpu.sync_copy(rows, acc.at[idxv.at[0]], add=True)

        return carry

    lax.fori_loop(0, NIT, step, 0)
    plsc.subcore_barrier()
    pltpu.sync_copy(acc.at[pl.ds(r0, NP_SUB)],
                    out.at[pl.ds(c * NPAD + r0, NP_SUB)])


def _sc_cnt(idx, zeros, ones):
    return pl.kernel(
        _sc_cnt_body,
        out_type=jax.ShapeDtypeStruct((2 * NPAD, H), jnp.float32),
        mesh=_mesh(),
        scratch_types=[
            pltpu.VMEM((1, CH), jnp.int32),
            pltpu.VMEM((CH, H), jnp.float32),
            pltpu.VMEM_SHARED((NPAD, H), jnp.float32),
        ],
    )(idx, zeros, ones)


def _sc_build_a_body(src, dst, za, out, sv, dv, posv, valv, acc):
    c = lax.axis_index("c")
    s = lax.axis_index("s")
    w = s * 2 + c
    r0 = s * (ASZ // 16)
    pltpu.sync_copy(za.at[pl.ds(r0, ASZ // 16)], acc.at[pl.ds(r0, ASZ // 16)])
    plsc.subcore_barrier()

    def step(i, carry):
        ch = i * NW + w

        @pl.when(ch < NCHUNK)
        def _():
            base = ch * CH
            pltpu.sync_copy(src.at[pl.ds(base, CH)], sv.at[0])
            pltpu.sync_copy(dst.at[pl.ds(base, CH)], dv.at[0])
            for j in range(CH // 16):
                s16 = sv[0, pl.ds(j * 16, 16)]
                d16 = dv[0, pl.ds(j * 16, 16)]
                srcm = s16 % NPG
                dstm = d16 % NPG
                same = (s16 - srcm) == (d16 - dstm)
                row = jnp.where(same, s16 - srcm,
                                jnp.zeros((16,), jnp.int32)) + srcm
                val = jnp.where(same, jnp.ones((16,), jnp.float32),
                                jnp.zeros((16,), jnp.float32))
                posv[0, pl.ds(j * 16, 16)] = row * NPG + dstm
                valv[0, pl.ds(j * 16, 16)] = val
            pltpu.sync_copy(valv.at[0], acc.at[posv.at[0]], add=True)

        return carry

    lax.fori_loop(0, NIT, step, 0)
    plsc.subcore_barrier()
    pltpu.sync_copy(acc.at[pl.ds(r0, ASZ // 16)],
                    out.at[pl.ds(c * ASZ + r0, ASZ // 16)])


def _sc_build_a(src, dst, za):
    return pl.kernel(
        _sc_build_a_body,
        out_type=jax.ShapeDtypeStruct((2 * ASZ,), jnp.float32),
        mesh=_mesh(),
        scratch_types=[
            pltpu.VMEM((1, CH), jnp.int32),
            pltpu.VMEM((1, CH), jnp.int32),
            pltpu.VMEM((1, CH), jnp.int32),
            pltpu.VMEM((1, CH), jnp.float32),
            pltpu.VMEM_SHARED((ASZ,), jnp.float32),
        ],
    )(src, dst, za)


def _onehot_rows(ids, width):
    cols = lax.broadcasted_iota(jnp.int32, (1, width), 1)
    return (ids[:, None] == cols).astype(jnp.float32)


def _rwse_body(a0, a1, xids, atom, rw_w, rw_b, out):
    a = a0[0] + a1[0]
    deg = jnp.sum(a, axis=1, keepdims=True)
    p = a / jnp.maximum(deg, 1.0)
    ri = lax.broadcasted_iota(jnp.int32, (NPG, NPG), 0)
    ci = lax.broadcasted_iota(jnp.int32, (NPG, NPG), 1)
    eye = (ri == ci).astype(jnp.float32)
    mt = p
    diags = []
    for _ in range(RWSE_STEPS):
        diags.append(jnp.sum(mt * eye, axis=1, keepdims=True))
        mt = jnp.dot(mt, p, preferred_element_type=jnp.float32)
    rw = jnp.concatenate(diags, axis=1)
    rwf = jnp.maximum(jnp.dot(rw, rw_w[...], preferred_element_type=jnp.float32)
                      + rw_b[...], 0.0)
    oh = _onehot_rows(xids[0, 0, :], VOCAB)
    out[0] = rwf + jnp.dot(oh, atom[...], preferred_element_type=jnp.float32)


def _tc_rwse(a0, a1, xids, atom, rw_w, rw_b):
    return pl.pallas_call(
        _rwse_body,
        grid=(G,),
        in_specs=[
            pl.BlockSpec((1, NPG, NPG), lambda i: (i, 0, 0)),
            pl.BlockSpec((1, NPG, NPG), lambda i: (i, 0, 0)),
            pl.BlockSpec((1, 1, NPG), lambda i: (i, 0, 0)),
            pl.BlockSpec((VOCAB, H), lambda i: (0, 0)),
            pl.BlockSpec((RWSE_STEPS, H), lambda i: (0, 0)),
            pl.BlockSpec((1, H), lambda i: (0, 0)),
        ],
        out_specs=pl.BlockSpec((1, NPG, H), lambda i: (i, 0, 0)),
        out_shape=jax.ShapeDtypeStruct((G, NPG, H), jnp.float32),
    )(a0, a1, xids, atom, rw_w, rw_b).reshape(N, H)


def _cinv_body(c0, c1, out):
    out[...] = 1.0 / jnp.maximum(c0[...] + c1[...], 1.0)


def _tc_cinv(c0, c1):
    return pl.pallas_call(
        _cinv_body,
        grid=(5,),
        in_specs=[pl.BlockSpec((N // 5, H), lambda i: (i, 0))] * 2,
        out_specs=pl.BlockSpec((N // 5, H), lambda i: (i, 0)),
        out_shape=jax.ShapeDtypeStruct((N, H), jnp.float32),
    )(c0, c1)


def _mlp(v, w1, b1, w2, b2):
    a = jnp.maximum(jnp.dot(v, w1, preferred_element_type=jnp.float32) + b1, 0.0)
    return jnp.dot(a, w2, preferred_element_type=jnp.float32) + b2


def _make_layer_body(first, last):
    def body(*refs):
        if first:
            (h_ref, ea1_ref, ea2_ref, bond_ref,
             w1r, b1r, w2r, b2r, epsr, *outs) = refs
            h3 = h_ref[...]
        else:
            (h_ref, hi_ref, gwf_ref, ea1_ref, ea2_ref, bond_ref,
             w1r, b1r, w2r, b2r, epsr, *outs) = refs
            h3 = h_ref[...] + hi_ref[...] + gwf_ref[...]
        if last:
            hi_out, h6r_out = outs
        elif first:
            (hi_out,) = outs
        else:
            hi_out, h_out = outs
            h_out[...] = h3
        bond = bond_ref[...]
        ea1 = jnp.dot(_onehot_rows(ea1_ref[0, 0, :] - 1, EDIM), bond,
                      preferred_element_type=jnp.float32).reshape(B_SG, K_SUB - 1, H)
        ea2 = jnp.dot(_onehot_rows(ea2_ref[0, 0, :] - 1, EDIM), bond,
                      preferred_element_type=jnp.float32).reshape(B_SG, K_SUB - 1, H)
        root = h3[:, 0:1, :]
        leaves = h3[:, 1:, :]
        eps1 = 1.0 + epsr[0, 0]
        star_root = eps1 * h3[:, 0, :] + jnp.sum(
            jnp.maximum(leaves + ea2, 0.0), axis=1)
        star_leaf = eps1 * leaves + jnp.maximum(root + ea1, 0.0)
        w1, b1, w2, b2 = w1r[...], b1r[...], w2r[...], b2r[...]
        hi_root = _mlp(star_root, w1, b1, w2, b2)
        hi_leaf = _mlp(star_leaf.reshape(B_SG * (K_SUB - 1), H),
                       w1, b1, w2, b2).reshape(B_SG, K_SUB - 1, H)
        hi_out[:, 0:1, :] = hi_root[:, None, :]
        hi_out[:, 1:, :] = hi_leaf
        if last:
            h6r_out[0] = h3[:, 0, :] + hi_root

    return body


def _tc_layer(h3, hi3, gwf3, ea1_ids, ea2_ids, bond, w1, b1, w2, b2, eps1,
              first, last):
    blk3 = pl.BlockSpec((B_SG, K_SUB, H), lambda i: (i, 0, 0))
    ids_spec = pl.BlockSpec((1, 1, B_SG * (K_SUB - 1)), lambda i: (i, 0, 0))
    w_specs = [
        pl.BlockSpec((EDIM, H), lambda i: (0, 0)),
        pl.BlockSpec((H, H), lambda i: (0, 0)),
        pl.BlockSpec((1, H), lambda i: (0, 0)),
        pl.BlockSpec((H, H), lambda i: (0, 0)),
        pl.BlockSpec((1, H), lambda i: (0, 0)),
        pl.BlockSpec(memory_space=pltpu.SMEM),
    ]
    hi_shape = jax.ShapeDtypeStruct((S, K_SUB, H), jnp.float32)
    if first:
        in_specs = [blk3, ids_spec, ids_spec] + w_specs
        args = (h3, ea1_ids, ea2_ids, bond, w1, b1, w2, b2, eps1)
        out_specs, out_shape = (blk3,), (hi_shape,)
    else:
        in_specs = [blk3, blk3, blk3, ids_spec, ids_spec] + w_specs
        args = (h3, hi3, gwf3, ea1_ids, ea2_ids, bond, w1, b1, w2, b2, eps1)
        if last:
            out_specs = (blk3, pl.BlockSpec((1, B_SG, H), lambda i: (i, 0, 0)))
            out_shape = (hi_shape,
                         jax.ShapeDtypeStruct((NBLK, B_SG, H), jnp.float32))
        else:
            out_specs, out_shape = (blk3, blk3), (hi_shape, hi_shape)
    return pl.pallas_call(
        _make_layer_body(first, last),
        grid=(NBLK,),
        in_specs=in_specs,
        out_specs=list(out_specs),
        out_shape=list(out_shape),
    )(*args)


def _gpool_body(g0, g1, cinv, out):
    out[...] = (g0[...] + g1[...]) * cinv[...]


def _tc_gpool(g0, g1, cinv):
    return pl.pallas_call(
        _gpool_body,
        grid=(5,),
        in_specs=[pl.BlockSpec((N // 5, H), lambda i: (i, 0))] * 3,
        out_specs=pl.BlockSpec((N // 5, H), lambda i: (i, 0)),
        out_shape=jax.ShapeDtypeStruct((N, H), jnp.float32),
    )(g0, g1, cinv)


def _me_body(ge, ids_ref, bond, out):
    ea = jnp.dot(_onehot_rows(ids_ref[0, 0, :] - 1, EDIM), bond[...],
                 preferred_element_type=jnp.float32)
    out[...] = jnp.maximum(ge[...] + ea, 0.0)


def _tc_me(ge, me_ids, bond):
    blk = E // NBLK
    return pl.pallas_call(
        _me_body,
        grid=(NBLK,),
        in_specs=[
            pl.BlockSpec((blk, H), lambda i: (i, 0)),
            pl.BlockSpec((1, 1, blk), lambda i: (i, 0, 0)),
            pl.BlockSpec((EDIM, H), lambda i: (0, 0)),
        ],
        out_specs=pl.BlockSpec((blk, H), lambda i: (i, 0)),
        out_shape=jax.ShapeDtypeStruct((E, H), jnp.float32),
    )(ge, me_ids, bond)


def _gw_body(g0, g1, wg, bg, out):
    out[...] = jnp.dot(g0[...] + g1[...], wg[...],
                       preferred_element_type=jnp.float32) + bg[...]


def _tc_gw(g0, g1, wg, bg):
    return pl.pallas_call(
        _gw_body,
        grid=(5,),
        in_specs=[
            pl.BlockSpec((N // 5, H), lambda i: (i, 0)),
            pl.BlockSpec((N // 5, H), lambda i: (i, 0)),
            pl.BlockSpec((H, H), lambda i: (0, 0)),
            pl.BlockSpec((1, H), lambda i: (0, 0)),
        ],
        out_specs=pl.BlockSpec((N // 5, H), lambda i: (i, 0)),
        out_shape=jax.ShapeDtypeStruct((N, H), jnp.float32),
    )(g0, g1, wg, bg)


def _layer_norm(v, g, b):
    mu = jnp.mean(v, axis=1, keepdims=True)
    var = jnp.mean((v - mu) ** 2, axis=1, keepdims=True)
    return (v - mu) / jnp.sqrt(var + 1e-5) * g + b


def _readout_body(h6r, gw6, lp, alpha, *rest):
    (l1g_a, l1b_a, wqkv_a, bqkv_a, wo_a, bo_a, l2g_a, l2b_a,
     wf1_a, bf1_a, wf2_a, bf2_a,
     l1g_b, l1b_b, wqkv_b, bqkv_b, wo_b, bo_b, l2g_b, l2b_b,
     wf1_b, bf1_b, wf2_b, bf2_b, ro_g, ro_b, out) = rest
    bn = h6r.shape[0]
    g = gw6[...]
    t0 = h6r[:, 0, :] + g
    t1 = h6r[:, 1, :] + g
    al = alpha[0, 0]
    lpv = lp[0]
    lp0 = al * lpv[:, 0:1]
    lp1 = al * lpv[:, 1:2]
    hri = lax.broadcasted_iota(jnp.int32, (H, HEADS), 0)
    hci = lax.broadcasted_iota(jnp.int32, (H, HEADS), 1)
    hs = (hri // DH == hci).astype(jnp.float32)
    he = hs.T
    inv = 1.0 / np.sqrt(DH)
    for (l1g, l1b, wqkv, bqkv, wo, bo, l2g, l2b, wf1, bf1, wf2, bf2) in (
        (l1g_a, l1b_a, wqkv_a, bqkv_a, wo_a, bo_a, l2g_a, l2b_a,
         wf1_a, bf1_a, wf2_a, bf2_a),
        (l1g_b, l1b_b, wqkv_b, bqkv_b, wo_b, bo_b, l2g_b, l2b_b,
         wf1_b, bf1_b, wf2_b, bf2_b),
    ):
        xn0 = _layer_norm(t0, l1g[...], l1b[...])
        xn1 = _layer_norm(t1, l1g[...], l1b[...])
        qkv0 = jnp.dot(xn0, wqkv[...], preferred_element_type=jnp.float32) + bqkv[...]
        qkv1 = jnp.dot(xn1, wqkv[...], preferred_element_type=jnp.float32) + bqkv[...]
        q0, k0, v0 = qkv0[:, :H], qkv0[:, H:2 * H], qkv0[:, 2 * H:]
        q1, k1, v1 = qkv1[:, :H], qkv1[:, H:2 * H], qkv1[:, 2 * H:]

        def scores(q, k, lpj):
            return jnp.dot(q * k, hs, preferred_element_type=jnp.float32) * inv + lpj

        s00 = scores(q0, k0, lp0)
        s01 = scores(q0, k1, lp1)
        s10 = scores(q1, k0, lp0)
        s11 = scores(q1, k1, lp1)

        def soft(sa, sb):
            m = jnp.maximum(sa, sb)
            ea = jnp.exp(sa - m)
            eb = jnp.exp(sb - m)
            d = ea + eb
            return ea / d, eb / d

        w00, w01 = soft(s00, s01)
        w10, w11 = soft(s10, s11)

        def expand(w):
            return jnp.dot(w, he, preferred_element_type=jnp.float32)

        o0 = expand(w00) * v0 + expand(w01) * v1
        o1 = expand(w10) * v0 + expand(w11) * v1
        t0 = t0 + jnp.dot(o0, wo[...], preferred_element_type=jnp.float32) + bo[...]
        t1 = t1 + jnp.dot(o1, wo[...], preferred_element_type=jnp.float32) + bo[...]
        xn20 = _layer_norm(t0, l2g[...], l2b[...])
        xn21 = _layer_norm(t1, l2g[...], l2b[...])
        t0 = t0 + jnp.dot(jax.nn.gelu(
            jnp.dot(xn20, wf1[...], preferred_element_type=jnp.float32) + bf1[...]),
            wf2[...], preferred_element_type=jnp.float32) + bf2[...]
        t1 = t1 + jnp.dot(jax.nn.gelu(
            jnp.dot(xn21, wf1[...], preferred_element_type=jnp.float32) + bf1[...]),
            wf2[...], preferred_element_type=jnp.float32) + bf2[...]
    ne = _layer_norm((t0 + t1) * 0.5, ro_g[...], ro_b[...])
    out[0] = jnp.sum(ne.reshape(bn // NPG, NPG, H), axis=1)


def _tc_readout(h6r3, gw6, lp3, alpha, ro_params, ro_g, ro_b):
    bn = 1000
    ngrid = N // bn
    full = lambda shp: pl.BlockSpec(shp, lambda i: tuple(0 for _ in shp))
    in_specs = [
        pl.BlockSpec((bn, M_SUB, H), lambda i: (i, 0, 0)),
        pl.BlockSpec((bn, H), lambda i: (i, 0)),
        pl.BlockSpec((1, bn, M_SUB), lambda i: (i, 0, 0)),
        pl.BlockSpec(memory_space=pltpu.SMEM),
    ]
    args = [h6r3, gw6, lp3, alpha]
    for rl in ro_params:
        for nm, shp in (("ln1_g", (1, H)), ("ln1_b", (1, H)),
                        ("Wqkv", (H, 3 * H)), ("bqkv", (1, 3 * H)),
                        ("Wo", (H, H)), ("bo", (1, H)),
                        ("ln2_g", (1, H)), ("ln2_b", (1, H)),
                        ("Wf1", (H, FFN)), ("bf1", (1, FFN)),
                        ("Wf2", (FFN, H)), ("bf2", (1, H))):
            in_specs.append(full(shp))
            args.append(rl[nm].reshape(shp))
    in_specs += [full((1, H)), full((1, H))]
    args += [ro_g.reshape(1, H), ro_b.reshape(1, H)]
    return pl.pallas_call(
        _readout_body,
        grid=(ngrid,),
        in_specs=in_specs,
        out_specs=pl.BlockSpec((1, bn // NPG, H), lambda i: (i, 0, 0)),
        out_shape=jax.ShapeDtypeStruct((ngrid, bn // NPG, H), jnp.float32),
    )(*args).reshape(G, H)


def kernel(x, edge_attr, edge_index, ptr, batch, nodes_sampled, log_probs, params):
    p = params
    nid = nodes_sampled.reshape(-1)
    nid2 = nid.reshape(NCHUNK, 1, CH)
    ea_ids = edge_attr[:, 0]
    n_star = S * (K_SUB - 1)
    ea1_ids = ea_ids[:n_star].reshape(NBLK, 1, n_star // NBLK)
    ea2_flat = jnp.concatenate([ea_ids[n_star:], ea_ids[:n_star - (E - n_star)]])
    ea2_ids = ea2_flat.reshape(NBLK, 1, n_star // NBLK)
    me_ids = ea_ids.reshape(NBLK, 1, E // NBLK)
    src = edge_index[0]
    dst = edge_index[1]
    src2 = src.reshape(NCHUNK, 1, CH)
    dst2 = dst.reshape(NCHUNK, 1, CH)
    xids = x[:, 0].reshape(G, 1, NPG)
    lp3 = log_probs.reshape(N // 100, 100, M_SUB).reshape(10, 1000, M_SUB)
    zeros_nh = jnp.zeros((NPAD, H), jnp.float32)
    zeros_a = jnp.zeros((ASZ,), jnp.float32)
    ones_ch = jnp.ones((CH, H), jnp.float32)
    bond = p['bond_emb']

    ap = _sc_build_a(src, dst, zeros_a)
    cntp = _sc_cnt(nid, zeros_nh, ones_ch)
    cinv = _tc_cinv(cntp[:N], cntp[NPAD:NPAD + N])
    table0 = _tc_rwse(ap[:AN * NPG].reshape(G, NPG, NPG),
                      ap[ASZ:ASZ + AN * NPG].reshape(G, NPG, NPG),
                      xids, p['atom_emb'],
                      p['rwse_W'], p['rwse_b'].reshape(1, H))
    h3 = _sc_gather(table0, nid2).reshape(S, K_SUB, H)

    hi3 = None
    gwf3 = None
    h6r = None
    gw = None
    for l in range(N_LAYERS):
        lyr = p['layers'][l]
        eps1 = lyr['eps'].reshape(1, 1)
        first = (l == 0)
        last = (l == N_LAYERS - 1)
        outs = _tc_layer(h3, hi3, gwf3, ea1_ids, ea2_ids, bond,
                         lyr['W1'], lyr['b1'].reshape(1, H),
                         lyr['W2'], lyr['b2'].reshape(1, H), eps1, first, last)
        if first:
            (hi_l,) = outs
        elif last:
            hi_l, h6r = outs
        else:
            hi_l, h3 = outs
        gsp = _sc_scatter(hi_l.reshape(R, H), nid2, zeros_nh)
        gpool = _tc_gpool(gsp[:N], gsp[NPAD:NPAD + N], cinv)
        gep = _sc_gather(gpool, src2)
        me = _tc_me(gep, me_ids, bond)
        gmp = _sc_scatter(me, dst2, zeros_nh)
        gw = _tc_gw(gmp[:N], gmp[NPAD:NPAD + N], lyr['Wg'],
                    lyr['bg'].reshape(1, H))
        if not last:
            gwf3 = _sc_gather(gw, nid2).reshape(S, K_SUB, H)
            hi3 = hi_l

    return _tc_readout(h6r.reshape(N, M_SUB, H), gw, lp3,

                       p['ht_alpha'].reshape(1, 1), p['readout'],
                       p['ro_g'], p['ro_b'])


# speedup vs baseline: 4.1942x; 1.0338x over previous
import functools

import jax
import jax.numpy as jnp
import numpy as np
from jax import lax
from jax.experimental import pallas as pl
from jax.experimental.pallas import tpu as pltpu, tpu_sc as plsc

N = 10000; E = 160000; H = 128; M_SUB = 2; K_SUB = 8
G = 100; NPG = 100; RWSE_STEPS = 16; HEADS = 4
N_LAYERS = 6; R_LAYERS = 2; VOCAB = 128; EDIM = 16; FFN = 4 * H
S = N * M_SUB
R = S * K_SUB
NW = 32
CH = 128
NCHUNK = R // CH
NIT = (NCHUNK + NW - 1) // NW
APAD = 112
AN = G * NPG
NPAD = 10240
NP_SUB = NPAD // 16
ASZ = 1 << 20
B_SG = 250
NBLK = S // B_SG
DH = H // HEADS

@functools.cache
def _mesh():
    return plsc.VectorSubcoreMesh(core_axis_name="c", subcore_axis_name="s",
                                  num_cores=2, num_subcores=16)


def _wid():
    return lax.axis_index("s") * 2 + lax.axis_index("c")


CPW = NCHUNK // NW
REM = NCHUNK - CPW * NW
NPAIR = (CPW + 2) // 2


def _load_idx_strip(idx2, idxv, w):
    base_ch = w * CPW
    pltpu.sync_copy(idx2.at[pl.ds(base_ch, CPW)], idxv.at[pl.ds(0, CPW)])

    @pl.when(w < REM)
    def _():
        pltpu.sync_copy(idx2.at[pl.ds(NW * CPW + w, 1)],
                        idxv.at[pl.ds(CPW, 1)])


def _idx_row(idxv, j):
    return idxv.at[j, 0]


def _gchunk(w, j):
    return jnp.where(j < CPW, w * CPW + j, NW * CPW + w)


def _sc_gather_body(table, idx2, out, idxv, b0, b1, s0, s1):
    w = _wid()
    _load_idx_strip(idx2, idxv, w)
    nch = CPW + jnp.where(w < REM, 1, 0)

    def issue(j, buf, sem):
        pltpu.async_copy(table.at[_idx_row(idxv, j)], buf, sem)

    def drain(j, buf, sem):
        pltpu.make_async_copy(table.at[pl.ds(0, CH)], buf, sem).wait()
        pltpu.sync_copy(buf, out.at[pl.ds(_gchunk(w, j) * CH, CH)])

    issue(jnp.int32(0), b0, s0)

    def pair(k, carry):
        j0 = 2 * k
        j1 = j0 + 1

        @pl.when(j1 < nch)
        def _():
            issue(j1, b1, s1)

        @pl.when(j0 < nch)
        def _():
            drain(j0, b0, s0)

        @pl.when(j0 + 2 < nch)
        def _():
            issue(j0 + 2, b0, s0)

        @pl.when(j1 < nch)
        def _():
            drain(j1, b1, s1)

        return carry

    lax.fori_loop(0, NPAIR, pair, 0)


def _sc_gather(table, idx2):
    return pl.kernel(
        _sc_gather_body,
        out_type=jax.ShapeDtypeStruct((R, H), jnp.float32),
        mesh=_mesh(),
        scratch_types=[
            pltpu.VMEM((CPW + 1, 1, CH), jnp.int32),
            pltpu.VMEM((CH, H), jnp.float32),
            pltpu.VMEM((CH, H), jnp.float32),
            pltpu.SemaphoreType.DMA,
            pltpu.SemaphoreType.DMA,
        ],
    )(table, idx2)


def _sc_scatter_body(vals, idx2, zeros, out, idxv, b0, b1, s0, s1, acc):
    c = lax.axis_index("c")
    s = lax.axis_index("s")
    w = s * 2 + c
    r0 = s * NP_SUB
    pltpu.sync_copy(zeros.at[pl.ds(r0, NP_SUB)], acc.at[pl.ds(r0, NP_SUB)])
    _load_idx_strip(idx2, idxv, w)
    plsc.subcore_barrier()
    nch = CPW + jnp.where(w < REM, 1, 0)

    def issue(j, buf, sem):
        pltpu.async_copy(vals.at[pl.ds(_gchunk(w, j) * CH, CH)], buf, sem)

    def drain(j, buf, sem):
        pltpu.make_async_copy(vals.at[pl.ds(0, CH)], buf, sem).wait()
        pltpu.sync_copy(buf, acc.at[_idx_row(idxv, j)], add=True)

    issue(jnp.int32(0), b0, s0)

    def pair(k, carry):
        j0 = 2 * k
        j1 = j0 + 1

        @pl.when(j1 < nch)
        def _():
            issue(j1, b1, s1)

        @pl.when(j0 < nch)
        def _():
            drain(j0, b0, s0)

        @pl.when(j0 + 2 < nch)
        def _():
            issue(j0 + 2, b0, s0)

        @pl.when(j1 < nch)
        def _():
            drain(j1, b1, s1)

        return carry

    lax.fori_loop(0, NPAIR, pair, 0)
    plsc.subcore_barrier()
    pltpu.sync_copy(acc.at[pl.ds(r0, NP_SUB)],
                    out.at[pl.ds(c * NPAD + r0, NP_SUB)])


def _sc_scatter(vals, idx2, zeros):
    return pl.kernel(
        _sc_scatter_body,
        out_type=jax.ShapeDtypeStruct((2 * NPAD, H), jnp.float32),
        mesh=_mesh(),
        scratch_types=[
            pltpu.VMEM((CPW + 1, 1, CH), jnp.int32),
            pltpu.VMEM((CH, H), jnp.float32),
            pltpu.VMEM((CH, H), jnp.float32),
            pltpu.SemaphoreType.DMA,
            pltpu.SemaphoreType.DMA,
            pltpu.VMEM_SHARED((NPAD, H), jnp.float32),
        ],
    )(vals, idx2, zeros)


def _sc_cnt_body(idx, zeros, ones, out, idxv, rows, acc):
    c = lax.axis_index("c")
    s = lax.axis_index("s")
    w = s * 2 + c
    r0 = s * NP_SUB
    pltpu.sync_copy(zeros.at[pl.ds(r0, NP_SUB)], acc.at[pl.ds(r0, NP_SUB)])
    pltpu.sync_copy(ones, rows)
    plsc.subcore_barrier()

    def step(i, carry):
        ch = i * NW + w

        @pl.when(ch < NCHUNK)
        def _():
            base = ch * CH
            pltpu.sync_copy(idx.at[pl.ds(base, CH)], idxv.at[0])
            pltpu.sync_copy(rows, acc.at[idxv.at[0]], add=True)

        return carry

    lax.fori_loop(0, NIT, step, 0)
    plsc.subcore_barrier()
    pltpu.sync_copy(acc.at[pl.ds(r0, NP_SUB)],
                    out.at[pl.ds(c * NPAD + r0, NP_SUB)])


def _sc_cnt(idx, zeros, ones):
    return pl.kernel(
        _sc_cnt_body,
        out_type=jax.ShapeDtypeStruct((2 * NPAD, H), jnp.float32),
        mesh=_mesh(),
        scratch_types=[
            pltpu.VMEM((1, CH), jnp.int32),
            pltpu.VMEM((CH, H), jnp.float32),
            pltpu.VMEM_SHARED((NPAD, H), jnp.float32),
        ],
    )(idx, zeros, ones)


def _sc_build_a_body(src, dst, za, out, sv, dv, posv, valv, acc):
    c = lax.axis_index("c")
    s = lax.axis_index("s")
    w = s * 2 + c
    r0 = s * (ASZ // 16)
    pltpu.sync_copy(za.at[pl.ds(r0, ASZ // 16)], acc.at[pl.ds(r0, ASZ // 16)])
    plsc.subcore_barrier()

    def step(i, carry):
        ch = i * NW + w

        @pl.when(ch < NCHUNK)
        def _():
            base = ch * CH
            pltpu.sync_copy(src.at[pl.ds(base, CH)], sv.at[0])
            pltpu.sync_copy(dst.at[pl.ds(base, CH)], dv.at[0])
            for j in range(CH // 16):
                s16 = sv[0, pl.ds(j * 16, 16)]
                d16 = dv[0, pl.ds(j * 16, 16)]
                srcm = s16 % NPG
                dstm = d16 % NPG
                same = (s16 - srcm) == (d16 - dstm)
                row = jnp.where(same, s16 - srcm,
                                jnp.zeros((16,), jnp.int32)) + srcm
                val = jnp.where(same, jnp.ones((16,), jnp.float32),
                                jnp.zeros((16,), jnp.float32))
                posv[0, pl.ds(j * 16, 16)] = row * NPG + dstm
                valv[0, pl.ds(j * 16, 16)] = val
            pltpu.sync_copy(valv.at[0], acc.at[posv.at[0]], add=True)

        return carry

    lax.fori_loop(0, NIT, step, 0)
    plsc.subcore_barrier()
    pltpu.sync_copy(acc.at[pl.ds(r0, ASZ // 16)],
                    out.at[pl.ds(c * ASZ + r0, ASZ // 16)])


def _sc_build_a(src, dst, za):
    return pl.kernel(
        _sc_build_a_body,
        out_type=jax.ShapeDtypeStruct((2 * ASZ,), jnp.float32),
        mesh=_mesh(),
        scratch_types=[
            pltpu.VMEM((1, CH), jnp.int32),
            pltpu.VMEM((1, CH), jnp.int32),
            pltpu.VMEM((1, CH), jnp.int32),
            pltpu.VMEM((1, CH), jnp.float32),
            pltpu.VMEM_SHARED((ASZ,), jnp.float32),
        ],
    )(src, dst, za)


def _onehot_rows(ids, width):
    cols = lax.broadcasted_iota(jnp.int32, (1, width), 1)
    return (ids[:, None] == cols).astype(jnp.float32)


def _rwse_body(a0, a1, xids, atom, rw_w, rw_b, out):
    a = a0[...] + a1[...]
    deg = jnp.sum(a, axis=2, keepdims=True)
    p = a / jnp.maximum(deg, 1.0)
    ri = lax.broadcasted_iota(jnp.int32, (1, NPG, NPG), 1)
    ci = lax.broadcasted_iota(jnp.int32, (1, NPG, NPG), 2)
    eye = (ri == ci).astype(jnp.float32)
    mt = p
    diags = []
    for _ in range(RWSE_STEPS):
        diags.append(jnp.sum(mt * eye, axis=2, keepdims=True))
        mt = jax.lax.dot_general(mt, p, (((2,), (1,)), ((0,), (0,))),
                                 preferred_element_type=jnp.float32)
    rw = jnp.concatenate(diags, axis=2).reshape(N, RWSE_STEPS)
    rwf = jnp.maximum(jnp.dot(rw, rw_w[...], preferred_element_type=jnp.float32)
                      + rw_b[...], 0.0)
    oh = _onehot_rows(xids[:, 0], VOCAB)
    out[...] = rwf + jnp.dot(oh, atom[...], preferred_element_type=jnp.float32)


def _tc_rwse(a0, a1, xids, atom, rw_w, rw_b):
    full3 = pl.BlockSpec((G, NPG, NPG), lambda: (0, 0, 0))
    return pl.pallas_call(
        _rwse_body,
        in_specs=[
            full3,
            full3,
            pl.BlockSpec((N, 1), lambda: (0, 0)),
            pl.BlockSpec((VOCAB, H), lambda: (0, 0)),
            pl.BlockSpec((RWSE_STEPS, H), lambda: (0, 0)),
            pl.BlockSpec((1, H), lambda: (0, 0)),
        ],
        out_specs=pl.BlockSpec((N, H), lambda: (0, 0)),
        out_shape=jax.ShapeDtypeStruct((N, H), jnp.float32),
    )(a0, a1, xids, atom, rw_w, rw_b)


def _cinv_body(c0, c1, out):
    out[...] = 1.0 / jnp.maximum(c0[...] + c1[...], 1.0)


def _tc_cinv(c0, c1):
    return pl.pallas_call(
        _cinv_body,
        grid=(5,),
        in_specs=[pl.BlockSpec((N // 5, H), lambda i: (i, 0))] * 2,
        out_specs=pl.BlockSpec((N // 5, H), lambda i: (i, 0)),
        out_shape=jax.ShapeDtypeStruct((N, H), jnp.float32),
    )(c0, c1)


def _mlp(v, w1, b1, w2, b2):
    a = jnp.maximum(jnp.dot(v, w1, preferred_element_type=jnp.float32) + b1, 0.0)
    return jnp.dot(a, w2, preferred_element_type=jnp.float32) + b2


def _make_layer_body(first, last):
    def body(*refs):
        if first:
            (h_ref, ea1_ref, ea2_ref, bond_ref,
             w1r, b1r, w2r, b2r, epsr, *outs) = refs
            h3 = h_ref[...]
        else:
            (h_ref, hi_ref, gwf_ref, ea1_ref, ea2_ref, bond_ref,
             w1r, b1r, w2r, b2r, epsr, *outs) = refs
            h3 = h_ref[...] + hi_ref[...] + gwf_ref[...]
        if last:
            hi_out, h6r_out = outs
        elif first:
            (hi_out,) = outs
        else:
            hi_out, h_out = outs
            h_out[...] = h3
        bond = bond_ref[...]
        ea1 = jnp.dot(_onehot_rows(ea1_ref[0, 0, :] - 1, EDIM), bond,
                      preferred_element_type=jnp.float32).reshape(B_SG, K_SUB - 1, H)
        ea2 = jnp.dot(_onehot_rows(ea2_ref[0, 0, :] - 1, EDIM), bond,
                      preferred_element_type=jnp.float32).reshape(B_SG, K_SUB - 1, H)
        root = h3[:, 0:1, :]
        leaves = h3[:, 1:, :]
        eps1 = 1.0 + epsr[0, 0]
        star_root = eps1 * h3[:, 0, :] + jnp.sum(
            jnp.maximum(leaves + ea2, 0.0), axis=1)
        star_leaf = eps1 * leaves + jnp.maximum(root + ea1, 0.0)
        w1, b1, w2, b2 = w1r[...], b1r[...], w2r[...], b2r[...]
        hi_root = _mlp(star_root, w1, b1, w2, b2)
        hi_leaf = _mlp(star_leaf.reshape(B_SG * (K_SUB - 1), H),
                       w1, b1, w2, b2).reshape(B_SG, K_SUB - 1, H)
        hi_out[:, 0:1, :] = hi_root[:, None, :]
        hi_out[:, 1:, :] = hi_leaf
        if last:
            h6r_out[0] = h3[:, 0, :] + hi_root

    return body


def _tc_layer(h3, hi3, gwf3, ea1_ids, ea2_ids, bond, w1, b1, w2, b2, eps1,
              first, last):
    blk3 = pl.BlockSpec((B_SG, K_SUB, H), lambda i: (i, 0, 0))
    ids_spec = pl.BlockSpec((1, 1, B_SG * (K_SUB - 1)), lambda i: (i, 0, 0))
    w_specs = [
        pl.BlockSpec((EDIM, H), lambda i: (0, 0)),
        pl.BlockSpec((H, H), lambda i: (0, 0)),
        pl.BlockSpec((1, H), lambda i: (0, 0)),
        pl.BlockSpec((H, H), lambda i: (0, 0)),
        pl.BlockSpec((1, H), lambda i: (0, 0)),
        pl.BlockSpec(memory_space=pltpu.SMEM),
    ]
    hi_shape = jax.ShapeDtypeStruct((S, K_SUB, H), jnp.float32)
    if first:
        in_specs = [blk3, ids_spec, ids_spec] + w_specs
        args = (h3, ea1_ids, ea2_ids, bond, w1, b1, w2, b2, eps1)
        out_specs, out_shape = (blk3,), (hi_shape,)
    else:
        in_specs = [blk3, blk3, blk3, ids_spec, ids_spec] + w_specs
        args = (h3, hi3, gwf3, ea1_ids, ea2_ids, bond, w1, b1, w2, b2, eps1)
        if last:
            out_specs = (blk3, pl.BlockSpec((1, B_SG, H), lambda i: (i, 0, 0)))
            out_shape = (hi_shape,
                         jax.ShapeDtypeStruct((NBLK, B_SG, H), jnp.float32))
        else:
            out_specs, out_shape = (blk3, blk3), (hi_shape, hi_shape)
    return pl.pallas_call(
        _make_layer_body(first, last),
        grid=(NBLK,),
        in_specs=in_specs,
        out_specs=list(out_specs),
        out_shape=list(out_shape),
    )(*args)


def _gpool_body(g0, g1, cinv, out):
    out[...] = (g0[...] + g1[...]) * cinv[...]


def _tc_gpool(g0, g1, cinv):
    return pl.pallas_call(
        _gpool_body,
        grid=(5,),
        in_specs=[pl.BlockSpec((N // 5, H), lambda i: (i, 0))] * 3,
        out_specs=pl.BlockSpec((N // 5, H), lambda i: (i, 0)),
        out_shape=jax.ShapeDtypeStruct((N, H), jnp.float32),
    )(g0, g1, cinv)


def _me_body(ge, ids_ref, bond, out):
    ea = jnp.dot(_onehot_rows(ids_ref[0, 0, :] - 1, EDIM), bond[...],
                 preferred_element_type=jnp.float32)
    out[...] = jnp.maximum(ge[...] + ea, 0.0)


def _tc_me(ge, me_ids, bond):
    blk = E // NBLK
    return pl.pallas_call(
        _me_body,
        grid=(NBLK,),
        in_specs=[
            pl.BlockSpec((blk, H), lambda i: (i, 0)),
            pl.BlockSpec((1, 1, blk), lambda i: (i, 0, 0)),
            pl.BlockSpec((EDIM, H), lambda i: (0, 0)),
        ],
        out_specs=pl.BlockSpec((blk, H), lambda i: (i, 0)),
        out_shape=jax.ShapeDtypeStruct((E, H), jnp.float32),
    )(ge, me_ids, bond)


def _gw_body(g0, g1, wg, bg, out):
    out[...] = jnp.dot(g0[...] + g1[...], wg[...],
                       preferred_element_type=jnp.float32) + bg[...]


def _tc_gw(g0, g1, wg, bg):
    return pl.pallas_call(
        _gw_body,
        grid=(5,),
        in_specs=[
            pl.BlockSpec((N // 5, H), lambda i: (i, 0)),
            pl.BlockSpec((N // 5, H), lambda i: (i, 0)),
            pl.BlockSpec((H, H), lambda i: (0, 0)),
            pl.BlockSpec((1, H), lambda i: (0, 0)),
        ],
        out_specs=pl.BlockSpec((N // 5, H), lambda i: (i, 0)),
        out_shape=jax.ShapeDtypeStruct((N, H), jnp.float32),
    )(g0, g1, wg, bg)


def _layer_norm(v, g, b):
    mu = jnp.mean(v, axis=1, keepdims=True)
    var = jnp.mean((v - mu) ** 2, axis=1, keepdims=True)
    return (v - mu) / jnp.sqrt(var + 1e-5) * g + b


def _readout_body(h6r, gw6, lp, alpha, *rest):
    (l1g_a, l1b_a, wqkv_a, bqkv_a, wo_a, bo_a, l2g_a, l2b_a,
     wf1_a, bf1_a, wf2_a, bf2_a,
     l1g_b, l1b_b, wqkv_b, bqkv_b, wo_b, bo_b, l2g_b, l2b_b,
     wf1_b, bf1_b, wf2_b, bf2_b, ro_g, ro_b, out) = rest
    bn = h6r.shape[0]
    g = gw6[...]
    t0 = h6r[:, 0, :] + g
    t1 = h6r[:, 1, :] + g
    al = alpha[0, 0]
    lpv = lp[0]
    lp0 = al * lpv[:, 0:1]
    lp1 = al * lpv[:, 1:2]
    hri = lax.broadcasted_iota(jnp.int32, (H, HEADS), 0)
    hci = lax.broadcasted_iota(jnp.int32, (H, HEADS), 1)
    hs = (hri // DH == hci).astype(jnp.float32)
    he = hs.T
    inv = 1.0 / np.sqrt(DH)
    for (l1g, l1b, wqkv, bqkv, wo, bo, l2g, l2b, wf1, bf1, wf2, bf2) in (
        (l1g_a, l1b_a, wqkv_a, bqkv_a, wo_a, bo_a, l2g_a, l2b_a,
         wf1_a, bf1_a, wf2_a, bf2_a),
        (l1g_b, l1b_b, wqkv_b, bqkv_b, wo_b, bo_b, l2g_b, l2b_b,
         wf1_b, bf1_b, wf2_b, bf2_b),
    ):
        xn0 = _layer_norm(t0, l1g[...], l1b[...])
        xn1 = _layer_norm(t1, l1g[...], l1b[...])
        qkv0 = jnp.dot(xn0, wqkv[...], preferred_element_type=jnp.float32) + bqkv[...]
        qkv1 = jnp.dot(xn1, wqkv[...], preferred_element_type=jnp.float32) + bqkv[...]
        q0, k0, v0 = qkv0[:, :H], qkv0[:, H:2 * H], qkv0[:, 2 * H:]
        q1, k1, v1 = qkv1[:, :H], qkv1[:, H:2 * H], qkv1[:, 2 * H:]

        def scores(q, k, lpj):
            return jnp.dot(q * k, hs, preferred_element_type=jnp.float32) * inv + lpj

        s00 = scores(q0, k0, lp0)
        s01 = scores(q0, k1, lp1)
        s10 = scores(q1, k0, lp0)
        s11 = scores(q1, k1, lp1)

        def soft(sa, sb):
            m = jnp.maximum(sa, sb)
            ea = jnp.exp(sa - m)
            eb = jnp.exp(sb - m)
            d = ea + eb
            return ea / d, eb / d

        w00, w01 = soft(s00, s01)
        w10, w11 = soft(s10, s11)

        def expand(w):
            return jnp.dot(w, he, preferred_element_type=jnp.float32)

        o0 = expand(w00) * v0 + expand(w01) * v1
        o1 = expand(w10) * v0 + expand(w11) * v1
        t0 = t0 + jnp.dot(o0, wo[...], preferred_element_type=jnp.float32) + bo[...]
        t1 = t1 + jnp.dot(o1, wo[...], preferred_element_type=jnp.float32) + bo[...]
        xn20 = _layer_norm(t0, l2g[...], l2b[...])
        xn21 = _layer_norm(t1, l2g[...], l2b[...])
        t0 = t0 + jnp.dot(jax.nn.gelu(
            jnp.dot(xn20, wf1[...], preferred_element_type=jnp.float32) + bf1[...]),
            wf2[...], preferred_element_type=jnp.float32) + bf2[...]
        t1 = t1 + jnp.dot(jax.nn.gelu(
            jnp.dot(xn21, wf1[...], preferred_element_type=jnp.float32) + bf1[...]),
            wf2[...], preferred_element_type=jnp.float32) + bf2[...]
    ne = _layer_norm((t0 + t1) * 0.5, ro_g[...], ro_b[...])
    out[0] = jnp.sum(ne.reshape(bn // NPG, NPG, H), axis=1)


def _tc_readout(h6r3, gw6, lp3, alpha, ro_params, ro_g, ro_b):
    bn = 1000
    ngrid = N // bn
    full = lambda shp: pl.BlockSpec(shp, lambda i: tuple(0 for _ in shp))
    in_specs = [
        pl.BlockSpec((bn, M_SUB, H), lambda i: (i, 0, 0)),
        pl.BlockSpec((bn, H), lambda i: (i, 0)),
        pl.BlockSpec((1, bn, M_SUB), lambda i: (i, 0, 0)),
        pl.BlockSpec(memory_space=pltpu.SMEM),
    ]
    args = [h6r3, gw6, lp3, alpha]
    for rl in ro_params:
        for nm, shp in (("ln1_g", (1, H)), ("ln1_b", (1, H)),
                        ("Wqkv", (H, 3 * H)), ("bqkv", (1, 3 * H)),
                        ("Wo", (H, H)), ("bo", (1, H)),
                        ("ln2_g", (1, H)), ("ln2_b", (1, H)),
                        ("Wf1", (H, FFN)), ("bf1", (1, FFN)),
                        ("Wf2", (FFN, H)), ("bf2", (1, H))):
            in_specs.append(full(shp))
            args.append(rl[nm].reshape(shp))
    in_specs += [full((1, H)), full((1, H))]
    args += [ro_g.reshape(1, H), ro_b.reshape(1, H)]
    return pl.pallas_call(
        _readout_body,
        grid=(ngrid,),
        in_specs=in_specs,
        out_specs=pl.BlockSpec((1, bn // NPG, H), lambda i: (i, 0, 0)),
        out_shape=jax.ShapeDtypeStruct((ngrid, bn // NPG, H), jnp.float32),
    )(*args).reshape(G, H)


def kernel(x, edge_attr, edge_index, ptr, batch, nodes_sampled, log_probs, params):
    p = params
    nid = nodes_sampled.reshape(-1)
    nid2 = nid.reshape(NCHUNK, 1, CH)
    ea_ids = edge_attr[:, 0]
    n_star = S * (K_SUB - 1)
    ea1_ids = ea_ids[:n_star].reshape(NBLK, 1, n_star // NBLK)
    ea2_flat = jnp.concatenate([ea_ids[n_star:], ea_ids[:n_star - (E - n_star)]])
    ea2_ids = ea2_flat.reshape(NBLK, 1, n_star // NBLK)
    me_ids = ea_ids.reshape(NBLK, 1, E // NBLK)
    src = edge_index[0]
    dst = edge_index[1]
    src2 = src.reshape(NCHUNK, 1, CH)
    dst2 = dst.reshape(NCHUNK, 1, CH)
    xids = x
    lp3 = log_probs.reshape(N // 100, 100, M_SUB).reshape(10, 1000, M_SUB)
    zeros_nh = jnp.zeros((NPAD, H), jnp.float32)
    zeros_a = jnp.zeros((ASZ,), jnp.float32)
    ones_ch = jnp.ones((CH, H), jnp.float32)
    bond = p['bond_emb']

    ap = _sc_build_a(src, dst, zeros_a)
    cntp = _sc_cnt(nid, zeros_nh, ones_ch)
    cinv = _tc_cinv(cntp[:N], cntp[NPAD:NPAD + N])
    table0 = _tc_rwse(ap[:AN * NPG].reshape(G, NPG, NPG),
                      ap[ASZ:ASZ + AN * NPG].reshape(G, NPG, NPG),
                      xids, p['atom_emb'],
                      p['rwse_W'], p['rwse_b'].reshape(1, H))
    h3 = _sc_gather(table0, nid2).reshape(S, K_SUB, H)

    hi3 = None
    gwf3 = None
    h6r = None
    gw = None
    for l in range(N_LAYERS):
        lyr = p['layers'][l]
        eps1 = lyr['eps'].reshape(1, 1)
        first = (l == 0)
        last = (l == N_LAYERS - 1)
        outs = _tc_layer(h3, hi3, gwf3, ea1_ids, ea2_ids, bond,
                         lyr['W1'], lyr['b1'].reshape(1, H),
                         lyr['W2'], lyr['b2'].reshape(1, H), eps1, first, last)
        if first:
            (hi_l,) = outs
        elif last:
            hi_l, h6r = outs
        else:
            hi_l, h3 = outs
        gsp = _sc_scatter(hi_l.reshape(R, H), nid2, zeros_nh)
        gpool = _tc_gpool(gsp[:N], gsp[NPAD:NPAD + N], cinv)
        gep = _sc_gather(gpool, src2)
        me = _tc_me(gep, me_ids, bond)
        gmp = _sc_scatter(me, dst2, zeros_nh)
        gw = _tc_gw(gmp[:N], gmp[NPAD:NPAD + N], lyr['Wg'],
                    lyr['bg'].reshape(1, H))
        if not last:
            gwf3 = _sc_gather(gw, nid2).reshape(S, K_SUB, H)
            hi3 = hi_l

    return _tc_readout(h6r.reshape(N, M_SUB, H), gw, lp3,

                       p['ht_alpha'].reshape(1, 1), p['readout'],
                       p['ro_g'], p['ro_b'])
